# Initial kernel scaffold; baseline (speedup 1.0000x reference)
import jax
import jax.numpy as jnp
from jax.experimental import pallas as pl


def kernel(img, snr, W_enc, b_enc, W_dec, b_dec, embed):
    raise NotImplementedError("write your pallas kernel here")


# speedup vs baseline: 3.0371x; 3.0371x over previous
import dataclasses
import functools

import jax
import jax.numpy as jnp
from jax.experimental import pallas as pl
from jax.experimental.pallas import tpu as pltpu
from jax.experimental.pallas import tpu_sc as plsc

_B = 4
_C_IN = 3
_HW = 384
_P = 16
_H = 24
_C_FEAT = 192
_K = 512
_SIGMA = 10.0
_M = _B * _H * _H
_D_PATCH = _C_IN * _P * _P
_N = _M * _C_FEAT // 2

_ENC_BM = _H * _H
_DEC_BM = _H * _H
_Q_BM = 2048
_NB = _N // _Q_BM
_GW = 128


def _enc_mm_body(p_ref, w_ref, b_ref, o_ref):
    z = (
        jnp.dot(
            p_ref[...].astype(jnp.bfloat16),
            w_ref[...].astype(jnp.bfloat16),
            preferred_element_type=jnp.float32,
        )
        + b_ref[...]
    )
    o_ref[...] = z.T


def _enc_matmul_t(p, w, b):
    return pl.pallas_call(
        _enc_mm_body,
        grid=(_B,),
        in_specs=[
            pl.BlockSpec((_ENC_BM, _D_PATCH), lambda b: (b, 0)),
            pl.BlockSpec((_D_PATCH, _C_FEAT), lambda b: (0, 0)),
            pl.BlockSpec((1, _C_FEAT), lambda b: (0, 0)),
        ],
        out_specs=pl.BlockSpec((_C_FEAT, _ENC_BM), lambda b: (b, 0)),
        out_shape=jax.ShapeDtypeStruct((_B * _C_FEAT, _H * _H), jnp.float32),
    )(p, w, b.reshape(1, _C_FEAT))


def _quant_body(xf_ref, tabb_ref, tab_ref, idx_ref, acc_ref, es_ref):
    i = pl.program_id(0)
    f0 = xf_ref[:, 0:1]
    f1 = xf_ref[:, 1:2]
    e2 = tab_ref[0:1, :]
    ne2 = tab_ref[1:2, :]
    prod = jnp.dot(
        xf_ref[...].astype(jnp.bfloat16),
        tabb_ref[0:2, :],
        preferred_element_type=jnp.float32,
    )
    nsf = (f0 * f0 + f1 * f1) * jnp.float32(-_SIGMA)
    t = (prod * jnp.float32(2.0 * _SIGMA) + nsf) + ne2
    m = jnp.max(t, axis=1, keepdims=True)
    p = jnp.exp(t - m)
    s_inv = 1.0 / jnp.sum(p, axis=1, keepdims=True)
    sm16 = (p * s_inv).astype(jnp.bfloat16)
    iota = jax.lax.broadcasted_iota(jnp.int32, (_Q_BM, _K), 1)
    masked = jnp.where(t == m, iota, _K)
    idx = jnp.min(masked, axis=1).astype(jnp.int32)
    onehot16 = jnp.where(
        idx[:, None] == jax.lax.broadcasted_iota(jnp.int32, (1, _K), 1),
        jnp.float32(1.0),
        jnp.float32(0.0),
    ).astype(jnp.bfloat16)
    ones_r = jnp.full((1, _Q_BM), 1.0, jnp.bfloat16)
    lik_part = jax.lax.dot_general(
        ones_r, sm16, (((1,), (0,)), ((), ())),
        preferred_element_type=jnp.float32,
    )
    cnt_part = jax.lax.dot_general(
        ones_r, onehot16, (((1,), (0,)), ((), ())),
        preferred_element_type=jnp.float32,
    )
    idx_ref[...] = idx

    @pl.when(i == 0)
    def _():
        acc_ref[...] = jnp.zeros_like(acc_ref)

    acc_ref[0:1, :] += lik_part
    acc_ref[1:2, :] += cnt_part

    @pl.when(i == _NB - 1)
    def _():
        es_ref[...] = (
            jnp.sum(acc_ref[1:2, :] * e2) * (1.0 / _N)
        ).reshape(1, 1)
        acc_ref[0:1, :] = acc_ref[0:1, :] * (1.0 / _N)


def _quantize_stats(xf, tabb, tab):
    return pl.pallas_call(
        _quant_body,
        grid=(_NB,),
        in_specs=[
            pl.BlockSpec((_Q_BM, 2), lambda i: (i, 0)),
            pl.BlockSpec((8, _K), lambda i: (0, 0)),
            pl.BlockSpec((8, _K), lambda i: (0, 0)),
        ],
        out_specs=[
            pl.BlockSpec((_Q_BM,), lambda i: (i,)),
            pl.BlockSpec((8, _K), lambda i: (0, 0)),
            pl.BlockSpec((1, 1), lambda i: (0, 0)),
        ],
        out_shape=[
            jax.ShapeDtypeStruct((_N,), jnp.int32),
            jax.ShapeDtypeStruct((8, _K), jnp.float32),
            jax.ShapeDtypeStruct((1, 1), jnp.float32),
        ],
    )(xf, tabb, tab)


_GD = 16


def _sc_gather(table, idx):
    n = idx.shape[0]
    nw = 32
    b_per_w = n // nw
    mesh = plsc.VectorSubcoreMesh(core_axis_name="c", subcore_axis_name="s")
    cp = pltpu.CompilerParams()
    fields = pltpu.CompilerParams.__dataclass_fields__
    if "needs_layout_passes" in fields:
        cp = dataclasses.replace(cp, needs_layout_passes=False)
    if "use_tc_tiling_on_sc" in fields:
        cp = dataclasses.replace(cp, use_tc_tiling_on_sc=False)

    @functools.partial(
        pl.kernel,
        out_type=jax.ShapeDtypeStruct((n, _GD), jnp.float32),
        mesh=mesh,
        compiler_params=cp,
        scratch_types=[
            pltpu.VMEM((b_per_w,), jnp.int32),
            pltpu.VMEM((b_per_w, _GD), jnp.float32),
            pltpu.SemaphoreType.DMA,
        ],
    )
    def gk(t_hbm, i_hbm, o_hbm, idx_v, rows_v, sem):
        wid = jax.lax.axis_index("s") * 2 + jax.lax.axis_index("c")
        base = wid * b_per_w
        pltpu.sync_copy(i_hbm.at[pl.ds(base, b_per_w)], idx_v)
        pltpu.async_copy(t_hbm.at[idx_v], rows_v, sem).wait()
        pltpu.sync_copy(rows_v, o_hbm.at[pl.ds(base, b_per_w)])

    return gk(table, idx)


def _dec_mm_body(q_ref, nz_ref, w_ref, b_ref, npw_ref, o_ref):
    y = q_ref[...] + npw_ref[...] * nz_ref[...]
    t = (
        jax.lax.dot_general(
            y.astype(jnp.bfloat16),
            w_ref[...].astype(jnp.bfloat16),
            (((0,), (0,)), ((), ())),
            preferred_element_type=jnp.float32,
        )
        + b_ref[...]
    )
    o_ref[...] = jax.nn.sigmoid(t)


def _dec_matmul(q4, nz4, w, b, npw):
    return pl.pallas_call(
        _dec_mm_body,
        grid=(_B,),
        in_specs=[
            pl.BlockSpec((_C_FEAT, _DEC_BM), lambda b: (b, 0)),
            pl.BlockSpec((_C_FEAT, _DEC_BM), lambda b: (b, 0)),
            pl.BlockSpec((_C_FEAT, _D_PATCH), lambda b: (0, 0)),
            pl.BlockSpec((1, _D_PATCH), lambda b: (0, 0)),
            pl.BlockSpec((1, 1), lambda b: (0, 0)),
        ],
        out_specs=pl.BlockSpec((_DEC_BM, _D_PATCH), lambda b: (b, 0)),
        out_shape=jax.ShapeDtypeStruct((_M, _D_PATCH), jnp.float32),
    )(q4, nz4, w, b.reshape(1, _D_PATCH), npw)


def kernel(img, snr, W_enc, b_enc, W_dec, b_dec, embed):
    p = (
        img.reshape(_B, _C_IN, _H, _P, _H, _P)
        .transpose(0, 2, 4, 1, 3, 5)
        .reshape(_M, _D_PATCH)
    )
    zt = _enc_matmul_t(p, W_enc, b_enc)
    xf = zt.reshape(_N, 2)

    ex = embed[:, 0]
    ey = embed[:, 1]
    e2 = ex * ex + ey * ey
    tab = jnp.zeros((8, _K), jnp.float32)
    tab = tab.at[0].set(e2)
    tab = tab.at[1].set(-_SIGMA * e2)
    tabb = jnp.zeros((8, _K), jnp.bfloat16).at[0:2, :].set(
        embed.T.astype(jnp.bfloat16)
    )
    idx, acc, es = _quantize_stats(xf, tabb, tab)
    likelihoods = acc[0]

    table16 = jnp.zeros((_K, _GD), jnp.float32).at[:, 0:2].set(embed)
    q = _sc_gather(table16, idx)[:, 0:2]

    es_s = es[0, 0]
    noise_pwr = jnp.sqrt(es_s * (10.0 ** (-jnp.asarray(snr, jnp.float32) / 10.0)) / 2.0)
    noise = jax.random.normal(jax.random.key(1), (_N, 2), jnp.float32)

    q4 = q.reshape(_B * _C_FEAT, _H * _H)
    nz4 = noise.reshape(_B * _C_FEAT, _H * _H)

    t = _dec_matmul(q4, nz4, W_dec, b_dec, noise_pwr.reshape(1, 1))
    output = (
        t.reshape(_B, _H, _H, _C_IN, _P, _P)
        .transpose(0, 3, 1, 4, 2, 5)
        .reshape(_B, _C_IN, _HW, _HW)
    )
    return output, likelihoods
